# Initial kernel scaffold; baseline (speedup 1.0000x reference)
import jax
import jax.numpy as jnp
from jax.experimental import pallas as pl


def kernel(idxs, vals, shared_emb, fm_first_emb, W1, b1, W2, b2, fcW, fcb):
    raise NotImplementedError("write your pallas kernel here")


# speedup vs baseline: 1.4703x; 1.4703x over previous
import functools

import jax
import jax.numpy as jnp
from jax import lax
from jax.experimental import pallas as pl
from jax.experimental.pallas import tpu as pltpu
from jax.experimental.pallas import tpu_sc as plsc

B = 16384
F = 26
D = 16
H1, H2 = 256, 128
NC, NS = 2, 16
NW = NC * NS
ROWS_PER_W = B // NW
CHUNK = 64
NCHUNK = ROWS_PER_W // CHUNK
NIDX = CHUNK * F
IDX_W = 128
NIDX_BLK = NIDX // IDX_W
CONSTS = 480


def _collapse_body(w1_ref, w2_ref, fc_ref, b1_ref, b2_ref, fcb_ref, u_ref, c_ref):
    fc1 = fc_ref[0:H2, :]
    w2f = jnp.dot(w2_ref[...], fc1, preferred_element_type=jnp.float32)
    u_ref[...] = jnp.dot(w1_ref[...], w2f, preferred_element_type=jnp.float32)
    c_ref[...] = (
        jnp.dot(b1_ref[...], w2f, preferred_element_type=jnp.float32)
        + jnp.dot(b2_ref[...], fc1, preferred_element_type=jnp.float32)
        + fcb_ref[...]
    )


def _collapse_weights(W1, W2, fcW, b1, b2, fcb):
    return pl.pallas_call(
        _collapse_body,
        out_shape=(
            jax.ShapeDtypeStruct((F * D, 1), jnp.float32),
            jax.ShapeDtypeStruct((1, 1), jnp.float32),
        ),
    )(W1, W2, fcW, b1.reshape(1, H1), b2.reshape(1, H2), fcb.reshape(1, 1))


_mesh = plsc.VectorSubcoreMesh(
    core_axis_name="c", subcore_axis_name="s", num_cores=NC, num_subcores=NS
)


@functools.partial(
    pl.kernel,
    out_type=jax.ShapeDtypeStruct((B,), jnp.float32),
    mesh=_mesh,
    scratch_types=[
        pltpu.VMEM((NIDX,), jnp.int32),
        pltpu.VMEM((NIDX + D,), jnp.float32),
        pltpu.VMEM((NIDX, D), jnp.float32),
        pltpu.VMEM((NIDX + D,), jnp.float32),
        pltpu.VMEM((CONSTS,), jnp.float32),
        pltpu.VMEM((ROWS_PER_W,), jnp.float32),
        pltpu.SemaphoreType.DMA,
        pltpu.SemaphoreType.DMA,
    ],
    compiler_params=pltpu.CompilerParams(needs_layout_passes=False,
                                         use_tc_tiling_on_sc=False),
)
def _sc_forward(idx_hbm, vals_hbm, emb_hbm, fm1_hbm, consts_hbm, out_hbm,
                idx_v, vals_v, rows_v, w1_v, consts_v, out_v, sem_e, sem_w):
    wid = lax.axis_index("s") * NC + lax.axis_index("c")
    base_row = wid * ROWS_PER_W

    pltpu.sync_copy(consts_hbm, consts_v)
    zero16 = jnp.zeros((D,), jnp.float32)
    vals_v[pl.ds(NIDX, D)] = zero16
    w1_v[pl.ds(NIDX, D)] = zero16
    a0 = consts_v[pl.ds(F * D, D)]
    a1 = consts_v[pl.ds(F * D + D, D)]
    g = consts_v[pl.ds(F * D + 2 * D, D)]
    cvec = consts_v[pl.ds(F * D + 3 * D, D)]
    lane_ids = lax.iota(jnp.int32, D)

    def chunk_body(ci, carry):
        off = base_row * F + ci * NIDX
        pltpu.sync_copy(idx_hbm.at[pl.ds(off, NIDX)], idx_v)
        pltpu.sync_copy(vals_hbm.at[pl.ds(off, NIDX)], vals_v.at[pl.ds(0, NIDX)])
        cps = []
        for j in range(NIDX_BLK):
            blk = idx_v.at[pl.ds(j * IDX_W, IDX_W)]
            cps.append(pltpu.async_copy(emb_hbm.at[blk],
                                        rows_v.at[pl.ds(j * IDX_W, IDX_W)], sem_e))
            cps.append(pltpu.async_copy(fm1_hbm.at[blk],
                                        w1_v.at[pl.ds(j * IDX_W, IDX_W)], sem_w))
        for cp in cps:
            cp.wait()

        def group_body(gi, carry2):
            gbase = gi * D

            def row_body(rr, logits):
                rbase = (gbase + rr) * F
                va = vals_v[pl.ds(rbase, D)]
                vb = vals_v[pl.ds(rbase + D, D)]
                s = jnp.zeros((D,), jnp.float32)
                q = jnp.zeros((D,), jnp.float32)
                t = jnp.zeros((D,), jnp.float32)
                for f in range(F):
                    e = rows_v[rbase + f]
                    v = va[f] if f < D else vb[f - D]
                    ev = e * v
                    s = s + ev
                    q = q + ev * ev
                    t = t + ev * consts_v[pl.ds(f * D, D)]
                w1a = w1_v[pl.ds(rbase, D)]
                w1b = w1_v[pl.ds(rbase + D, D)]
                tot = (t + w1a * va * a0 + w1b * vb * a1
                       + 0.5 * (s * s - q) * g + cvec)
                return jnp.where(lane_ids == rr, jnp.sum(tot), logits)

            acc = lax.fori_loop(0, D, row_body, jnp.zeros((D,), jnp.float32))
            out_v[pl.ds(ci * CHUNK + gbase, D)] = 1.0 / (1.0 + jnp.exp(-acc))
            return carry2

        return lax.fori_loop(0, CHUNK // D, group_body, carry)

    lax.fori_loop(0, NCHUNK, chunk_body, 0)
    pltpu.sync_copy(out_v, out_hbm.at[pl.ds(base_row, ROWS_PER_W)])


def kernel(idxs, vals, shared_emb, fm_first_emb, W1, b1, W2, b2, fcW, fcb):
    u2d, c2d = _collapse_weights(W1, W2, fcW, b1, b2, fcb)
    fc = fcW[:, 0]
    a_pad = jnp.zeros((2 * D,), jnp.float32).at[:F].set(fc[H2:H2 + F])
    g = fc[H2 + F:H2 + F + D]
    cvec = jnp.broadcast_to(c2d.reshape(1) * (1.0 / D), (D,))
    consts = jnp.concatenate([u2d.reshape(F * D), a_pad, g, cvec])
    out = _sc_forward(idxs.astype(jnp.int32).reshape(B * F), vals.reshape(B * F),
                      shared_emb, fm_first_emb.reshape(-1), consts)
    return out.reshape(B, 1)


# speedup vs baseline: 1.5063x; 1.0245x over previous
import functools

import jax
import jax.numpy as jnp
from jax import lax
from jax.experimental import pallas as pl
from jax.experimental.pallas import tpu as pltpu
from jax.experimental.pallas import tpu_sc as plsc

B = 16384
F = 26
D = 16
H1, H2 = 256, 128
NC, NS = 2, 16
NW = NC * NS
ROWS_PER_W = B // NW
CHUNK = 128
NCHUNK = ROWS_PER_W // CHUNK
NIDX = CHUNK * F
CONSTS = 480


def _collapse_body(w1_ref, w2_ref, fc_ref, b1_ref, b2_ref, fcb_ref, u_ref, c_ref):
    fc1 = fc_ref[0:H2, :]
    w2f = jnp.dot(w2_ref[...], fc1, preferred_element_type=jnp.float32)
    u_ref[...] = jnp.dot(w1_ref[...], w2f, preferred_element_type=jnp.float32)
    c_ref[...] = (
        jnp.dot(b1_ref[...], w2f, preferred_element_type=jnp.float32)
        + jnp.dot(b2_ref[...], fc1, preferred_element_type=jnp.float32)
        + fcb_ref[...]
    )


def _collapse_weights(W1, W2, fcW, b1, b2, fcb):
    return pl.pallas_call(
        _collapse_body,
        out_shape=(
            jax.ShapeDtypeStruct((F * D, 1), jnp.float32),
            jax.ShapeDtypeStruct((1, 1), jnp.float32),
        ),
    )(W1, W2, fcW, b1.reshape(1, H1), b2.reshape(1, H2), fcb.reshape(1, 1))


_mesh = plsc.VectorSubcoreMesh(
    core_axis_name="c", subcore_axis_name="s", num_cores=NC, num_subcores=NS
)


@functools.partial(
    pl.kernel,
    out_type=jax.ShapeDtypeStruct((B,), jnp.float32),
    mesh=_mesh,
    scratch_types=[
        pltpu.VMEM((2, F, CHUNK), jnp.int32),
        pltpu.VMEM((2, F, CHUNK), jnp.float32),
        pltpu.VMEM((2 * NIDX, D), jnp.float32),
        pltpu.VMEM((2 * NIDX,), jnp.float32),
        pltpu.VMEM((CONSTS,), jnp.float32),
        pltpu.VMEM((ROWS_PER_W,), jnp.float32),
        pltpu.SemaphoreType.DMA,
        pltpu.SemaphoreType.DMA,
        pltpu.SemaphoreType.DMA,
        pltpu.SemaphoreType.DMA,
    ],
    compiler_params=pltpu.CompilerParams(needs_layout_passes=False,
                                         use_tc_tiling_on_sc=False),
)
def _sc_forward(idxT_hbm, valsT_hbm, emb_hbm, fm1_hbm, consts_hbm, out_hbm,
                idx_v, vals_v, rows_v, w1_v, consts_v, out_v,
                sem_e0, sem_w0, sem_e1, sem_w1):
    wid = lax.axis_index("s") * NC + lax.axis_index("c")
    base_row = wid * ROWS_PER_W
    sems = ((sem_e0, sem_w0), (sem_e1, sem_w1))

    pltpu.sync_copy(consts_hbm, consts_v)
    a0 = consts_v[pl.ds(F * D, D)]
    a1 = consts_v[pl.ds(F * D + D, D)]
    g = consts_v[pl.ds(F * D + 2 * D, D)]
    cvec = consts_v[pl.ds(F * D + 3 * D, D)]
    lane_ids = lax.iota(jnp.int32, D)
    fclamp = jnp.minimum(lane_ids + D, F - 1)

    def stage(ci):
        p = ci % 2
        sem_e, sem_w = sems[p]
        col0 = base_row + ci * CHUNK
        pltpu.sync_copy(idxT_hbm.at[:, pl.ds(col0, CHUNK)], idx_v.at[p])
        pltpu.sync_copy(valsT_hbm.at[:, pl.ds(col0, CHUNK)], vals_v.at[p])
        cps = []
        for f in range(F):
            blk = idx_v.at[p, f]
            cps.append(pltpu.async_copy(
                emb_hbm.at[blk],
                rows_v.at[pl.ds(p * NIDX + f * CHUNK, CHUNK)], sem_e))
            cps.append(pltpu.async_copy(
                fm1_hbm.at[blk],
                w1_v.at[pl.ds(p * NIDX + f * CHUNK, CHUNK)], sem_w))
        return cps

    def compute(ci):
        p = ci % 2

        def group_body(gi, carry2):
            gbase = gi * D

            def row_body(rr, logits):
                rl = gbase + rr
                rsplat = jnp.full((D,), rl, jnp.int32)
                vva = plsc.load_gather(vals_v, [jnp.full((D,), p, jnp.int32),
                                                lane_ids, rsplat])
                vvb = plsc.load_gather(vals_v, [jnp.full((D,), p, jnp.int32),
                                                fclamp, rsplat])
                w1a = plsc.load_gather(w1_v, [p * NIDX + lane_ids * CHUNK + rl])
                w1b = plsc.load_gather(w1_v, [p * NIDX + fclamp * CHUNK + rl])
                s = jnp.zeros((D,), jnp.float32)
                q = jnp.zeros((D,), jnp.float32)
                t = jnp.zeros((D,), jnp.float32)
                for f in range(F):
                    e = rows_v[p * NIDX + f * CHUNK + rl]
                    v = vva[f] if f < D else vvb[f - D]
                    ev = e * v
                    s = s + ev
                    q = q + ev * ev
                    t = t + ev * consts_v[pl.ds(f * D, D)]
                tot = (t + w1a * vva * a0 + w1b * vvb * a1
                       + 0.5 * (s * s - q) * g + cvec)
                return jnp.where(lane_ids == rr, jnp.sum(tot), logits)

            acc = lax.fori_loop(0, D, row_body, jnp.zeros((D,), jnp.float32))
            out_v[pl.ds(ci * CHUNK + gbase, D)] = 1.0 / (1.0 + jnp.exp(-acc))
            return carry2

        lax.fori_loop(0, CHUNK // D, group_body, 0)

    cur = stage(0)
    for ci in range(NCHUNK):
        nxt = stage(ci + 1) if ci + 1 < NCHUNK else None
        for cp in cur:
            cp.wait()
        compute(ci)
        cur = nxt

    pltpu.sync_copy(out_v, out_hbm.at[pl.ds(base_row, ROWS_PER_W)])


def kernel(idxs, vals, shared_emb, fm_first_emb, W1, b1, W2, b2, fcW, fcb):
    u2d, c2d = _collapse_weights(W1, W2, fcW, b1, b2, fcb)
    fc = fcW[:, 0]
    a_pad = jnp.zeros((2 * D,), jnp.float32).at[:F].set(fc[H2:H2 + F])
    g = fc[H2 + F:H2 + F + D]
    cvec = jnp.broadcast_to(c2d.reshape(1) * (1.0 / D), (D,))
    consts = jnp.concatenate([u2d.reshape(F * D), a_pad, g, cvec])
    out = _sc_forward(idxs.astype(jnp.int32).T, vals.T,
                      shared_emb, fm_first_emb.reshape(-1), consts)
    return out.reshape(B, 1)
